# Initial kernel scaffold; baseline (speedup 1.0000x reference)
import jax
import jax.numpy as jnp
from jax.experimental import pallas as pl


def kernel(x, edge_index, edge_attr, u, batch, e_W1, e_b1, e_W2, e_b2, Mk, Mv, n_W1, n_b1, n_W2, n_b2, g_W1, g_b1, g_W2, g_b2):
    raise NotImplementedError("write your pallas kernel here")


# speedup vs baseline: 4.4868x; 4.4868x over previous
import jax
import jax.numpy as jnp
from jax import lax
from jax.experimental import pallas as pl
from jax.experimental.pallas import tpu as pltpu
from jax.experimental.pallas import tpu_sc as plsc

_NC = 2
_NS = 16
_NW = _NC * _NS
_LANES = 16
_CHUNK = 80


def _prep_body(x_ref, u_ref, w1s_ref, w1d_ref, w1u_ref, eb1_ref, xs_ref, xd_ref):
    c0 = jnp.dot(u_ref[...], w1u_ref[...], preferred_element_type=jnp.float32) + eb1_ref[...]
    xs_ref[...] = jnp.dot(x_ref[...], w1s_ref[...], preferred_element_type=jnp.float32) + c0
    xd_ref[...] = jnp.dot(x_ref[...], w1d_ref[...], preferred_element_type=jnp.float32)


def _prep(x, u2, w1s, w1d, w1u, eb1, bn):
    n, vin = x.shape
    h = w1s.shape[1]
    return pl.pallas_call(
        _prep_body,
        grid=(n // bn,),
        in_specs=[
            pl.BlockSpec((bn, vin), lambda i: (i, 0)),
            pl.BlockSpec(u2.shape, lambda i: (0, 0)),
            pl.BlockSpec(w1s.shape, lambda i: (0, 0)),
            pl.BlockSpec(w1d.shape, lambda i: (0, 0)),
            pl.BlockSpec(w1u.shape, lambda i: (0, 0)),
            pl.BlockSpec(eb1.shape, lambda i: (0, 0)),
        ],
        out_specs=[pl.BlockSpec((bn, h), lambda i: (i, 0)),
                   pl.BlockSpec((bn, h), lambda i: (i, 0))],
        out_shape=[jax.ShapeDtypeStruct((n, h), jnp.float32),
                   jax.ShapeDtypeStruct((n, h), jnp.float32)],
    )(x, u2, w1s, w1d, w1u, eb1)


def _sc_gather_add(xs, xd, row, col):
    e = row.shape[0]
    h = xs.shape[1]
    epw = e // _NW
    nchunks = epw // _CHUNK
    mesh = plsc.VectorSubcoreMesh(core_axis_name="c", subcore_axis_name="s")

    def body(xs_hbm, xd_hbm, row_hbm, col_hbm, s_hbm, idx_r, idx_c, a_v, b_v, sem):
        wid = lax.axis_index("s") * _NC + lax.axis_index("c")

        def chunk(k, carry):
            base = pl.multiple_of(wid * epw + k * _CHUNK, 8)
            pltpu.sync_copy(row_hbm.at[pl.ds(base, _CHUNK)], idx_r)
            pltpu.sync_copy(col_hbm.at[pl.ds(base, _CHUNK)], idx_c)
            ca = pltpu.async_copy(xs_hbm.at[idx_r], a_v, sem)
            cb = pltpu.async_copy(xd_hbm.at[idx_c], b_v, sem)
            ca.wait()
            cb.wait()

            def addrow(j, c2):
                for q in range(h // _LANES):
                    sl = pl.ds(q * _LANES, _LANES)
                    a_v[j, sl] = a_v[j, sl] + b_v[j, sl]
                return c2

            lax.fori_loop(0, _CHUNK, addrow, 0)
            pltpu.sync_copy(a_v, s_hbm.at[pl.ds(base, _CHUNK)])
            return carry

        lax.fori_loop(0, nchunks, chunk, 0)

    f = pl.kernel(
        body,
        out_type=jax.ShapeDtypeStruct((e, h), jnp.float32),
        mesh=mesh,
        scratch_types=[
            pltpu.VMEM((_CHUNK,), jnp.int32),
            pltpu.VMEM((_CHUNK,), jnp.int32),
            pltpu.VMEM((_CHUNK, h), jnp.float32),
            pltpu.VMEM((_CHUNK, h), jnp.float32),
            pltpu.SemaphoreType.DMA,
        ],
    )
    return f(xs, xd, row, col)


def _edge_body(s_ref, ea_ref, w1e_ref, w2_ref, eb2_ref, ne_ref, esum_ref, acc_ref):
    i = pl.program_id(0)
    pre = s_ref[...] + jnp.dot(ea_ref[...], w1e_ref[...], preferred_element_type=jnp.float32)
    hh = jnp.maximum(pre, 0.0)
    ne = jnp.dot(hh, w2_ref[...], preferred_element_type=jnp.float32) + eb2_ref[...]
    ne_ref[...] = ne

    @pl.when(i == 0)
    def _():
        acc_ref[...] = jnp.zeros_like(acc_ref)

    acc_ref[...] += jnp.sum(ne, axis=0, keepdims=True)

    @pl.when(i == pl.num_programs(0) - 1)
    def _():
        esum_ref[...] = acc_ref[...]


def _edge_mlp(s_e, ea, w1e, w2, eb2, be):
    e, h = s_e.shape
    ein = ea.shape[1]
    return pl.pallas_call(
        _edge_body,
        grid=(e // be,),
        in_specs=[
            pl.BlockSpec((be, h), lambda i: (i, 0)),
            pl.BlockSpec((be, ein), lambda i: (i, 0)),
            pl.BlockSpec(w1e.shape, lambda i: (0, 0)),
            pl.BlockSpec(w2.shape, lambda i: (0, 0)),
            pl.BlockSpec(eb2.shape, lambda i: (0, 0)),
        ],
        out_specs=[pl.BlockSpec((be, h), lambda i: (i, 0)),
                   pl.BlockSpec((1, h), lambda i: (0, 0))],
        out_shape=[jax.ShapeDtypeStruct((e, h), jnp.float32),
                   jax.ShapeDtypeStruct((1, h), jnp.float32)],
        scratch_shapes=[pltpu.VMEM((1, h), jnp.float32)],
    )(s_e, ea, w1e, w2, eb2)


def _sc_scatter_add(ne, col, n):
    e, h = ne.shape
    epw = e // _NW
    nchunks = epw // _CHUNK
    rc = _CHUNK
    nrch = n // rc
    per = (nrch + _NS - 1) // _NS
    mesh = plsc.VectorSubcoreMesh(core_axis_name="c", subcore_axis_name="s")

    def body(ne_hbm, col_hbm, out_hbm, ne_v, idx_v, z_v, agg_sh, sem):
        cid = lax.axis_index("c")
        sid = lax.axis_index("s")
        wid = sid * _NC + cid

        def zrow(j, carry):
            for q in range(h // _LANES):
                z_v[j, pl.ds(q * _LANES, _LANES)] = jnp.zeros((_LANES,), jnp.float32)
            return carry

        lax.fori_loop(0, rc, zrow, 0)
        for t in range(per):
            cix = sid + _NS * t

            @pl.when(cix < nrch)
            def _():
                r0 = pl.multiple_of(cix * rc, 8)
                pltpu.sync_copy(z_v, agg_sh.at[pl.ds(r0, rc)])

        plsc.subcore_barrier()

        def chunk(k, carry):
            base = pl.multiple_of(wid * epw + k * _CHUNK, 8)
            pltpu.sync_copy(col_hbm.at[pl.ds(base, _CHUNK)], idx_v)
            pltpu.sync_copy(ne_hbm.at[pl.ds(base, _CHUNK)], ne_v)
            pltpu.sync_copy(ne_v, agg_sh.at[idx_v], add=True)
            return carry

        lax.fori_loop(0, nchunks, chunk, 0)
        plsc.subcore_barrier()
        for t in range(per):
            cix = sid + _NS * t

            @pl.when(cix < nrch)
            def _():
                r0 = pl.multiple_of(cix * rc, 8)
                pltpu.sync_copy(agg_sh.at[pl.ds(r0, rc)], z_v)
                pltpu.sync_copy(z_v, out_hbm.at[cid, pl.ds(r0, rc)])

    f = pl.kernel(
        body,
        out_type=jax.ShapeDtypeStruct((_NC, n, h), jnp.float32),
        mesh=mesh,
        scratch_types=[
            pltpu.VMEM((_CHUNK, h), jnp.float32),
            pltpu.VMEM((_CHUNK,), jnp.int32),
            pltpu.VMEM((rc, h), jnp.float32),
            pltpu.VMEM_SHARED((n, h), jnp.float32),
            pltpu.SemaphoreType.DMA,
        ],
    )
    return f(ne, col)


def _logits_body(x_ref, a0_ref, a1_ref, u_ref, mkx_ref, mka_ref, mku_ref, l_ref):
    agg = a0_ref[...] + a1_ref[...]
    l_ref[...] = (
        jnp.dot(x_ref[...], mkx_ref[...], preferred_element_type=jnp.float32)
        + jnp.dot(agg, mka_ref[...], preferred_element_type=jnp.float32)
        + jnp.dot(u_ref[...], mku_ref[...], preferred_element_type=jnp.float32)
    )


def _logits(x, a0, a1, u2, mkx, mka, mku, bn):
    n, vin = x.shape
    h = a0.shape[1]
    m = mkx.shape[1]
    return pl.pallas_call(
        _logits_body,
        grid=(n // bn,),
        in_specs=[
            pl.BlockSpec((bn, vin), lambda i: (i, 0)),
            pl.BlockSpec((bn, h), lambda i: (i, 0)),
            pl.BlockSpec((bn, h), lambda i: (i, 0)),
            pl.BlockSpec(u2.shape, lambda i: (0, 0)),
            pl.BlockSpec(mkx.shape, lambda i: (0, 0)),
            pl.BlockSpec(mka.shape, lambda i: (0, 0)),
            pl.BlockSpec(mku.shape, lambda i: (0, 0)),
        ],
        out_specs=pl.BlockSpec((bn, m), lambda i: (i, 0)),
        out_shape=jax.ShapeDtypeStruct((n, m), jnp.float32),
    )(x, a0, a1, u2, mkx, mka, mku)


def _stats_body(l_ref, m_ref, s_ref, macc_ref, sacc_ref):
    i = pl.program_id(0)

    @pl.when(i == 0)
    def _():
        macc_ref[...] = jnp.full_like(macc_ref, -1e30)
        sacc_ref[...] = jnp.zeros_like(sacc_ref)

    l = l_ref[...]
    bm = jnp.max(l, axis=0, keepdims=True)
    mold = macc_ref[...]
    mnew = jnp.maximum(mold, bm)
    sacc_ref[...] = sacc_ref[...] * jnp.exp(mold - mnew) + jnp.sum(
        jnp.exp(l - mnew), axis=0, keepdims=True)
    macc_ref[...] = mnew

    @pl.when(i == pl.num_programs(0) - 1)
    def _():
        m_ref[...] = macc_ref[...]
        s_ref[...] = sacc_ref[...]


def _softmax_stats(logits, bn):
    n, m = logits.shape
    return pl.pallas_call(
        _stats_body,
        grid=(n // bn,),
        in_specs=[pl.BlockSpec((bn, m), lambda i: (i, 0))],
        out_specs=[pl.BlockSpec((1, m), lambda i: (0, 0)),
                   pl.BlockSpec((1, m), lambda i: (0, 0))],
        out_shape=[jax.ShapeDtypeStruct((1, m), jnp.float32),
                   jax.ShapeDtypeStruct((1, m), jnp.float32)],
        scratch_shapes=[pltpu.VMEM((1, m), jnp.float32),
                        pltpu.VMEM((1, m), jnp.float32)],
    )(logits)


def _node_body(n_nodes, n_edges,
               l_ref, m_ref, s_ref, mv_ref, nw1_ref, nb1_ref, nw2_ref, nb2_ref,
               u_ref, esum_ref, gw1_ref, gb1_ref, gw2_ref, gb2_ref,
               nx_ref, nu_ref, acc_ref):
    i = pl.program_id(0)
    attn = jnp.exp(l_ref[...] - m_ref[...]) / s_ref[...]
    attn = attn / jnp.sum(attn, axis=1, keepdims=True)
    ao = jnp.dot(attn, mv_ref[...], preferred_element_type=jnp.float32)
    hn = jnp.maximum(
        jnp.dot(ao, nw1_ref[...], preferred_element_type=jnp.float32) + nb1_ref[...], 0.0)
    nx = jnp.dot(hn, nw2_ref[...], preferred_element_type=jnp.float32) + nb2_ref[...]
    nx_ref[...] = nx

    @pl.when(i == 0)
    def _():
        acc_ref[...] = jnp.zeros_like(acc_ref)

    acc_ref[...] += jnp.sum(nx, axis=0, keepdims=True)

    @pl.when(i == pl.num_programs(0) - 1)
    def _():
        node_agg = acc_ref[...] * (1.0 / n_nodes)
        edge_agg = esum_ref[...] * (1.0 / n_edges)
        gcat = jnp.concatenate([u_ref[...], node_agg, edge_agg], axis=1)
        gh = jnp.maximum(
            jnp.dot(gcat, gw1_ref[...], preferred_element_type=jnp.float32) + gb1_ref[...], 0.0)
        nu_ref[...] = jnp.dot(gh, gw2_ref[...], preferred_element_type=jnp.float32) + gb2_ref[...]


def _node_global_call(logits, mcol, scol, mv, nw1, nb1, nw2, nb2, u2, esum,
                      gw1, gb1, gw2, gb2, bn, n_edges):
    import functools
    n, m = logits.shape
    h = nw2.shape[1]
    body = functools.partial(_node_body, float(n), float(n_edges))
    full = lambda a: pl.BlockSpec(a.shape, lambda i: tuple(0 for _ in a.shape))
    return pl.pallas_call(
        body,
        grid=(n // bn,),
        in_specs=[
            pl.BlockSpec((bn, m), lambda i: (i, 0)),
            full(mcol), full(scol), full(mv), full(nw1), full(nb1),
            full(nw2), full(nb2), full(u2), full(esum), full(gw1),
            full(gb1), full(gw2), full(gb2),
        ],
        out_specs=[pl.BlockSpec((bn, h), lambda i: (i, 0)),
                   pl.BlockSpec((1, h), lambda i: (0, 0))],
        out_shape=[jax.ShapeDtypeStruct((n, h), jnp.float32),
                   jax.ShapeDtypeStruct((1, h), jnp.float32)],
        scratch_shapes=[pltpu.VMEM((1, h), jnp.float32)],
    )(logits, mcol, scol, mv, nw1, nb1, nw2, nb2, u2, esum, gw1, gb1, gw2, gb2)


def kernel(x, edge_index, edge_attr, u, batch,
           e_W1, e_b1, e_W2, e_b2, Mk, Mv,
           n_W1, n_b1, n_W2, n_b2, g_W1, g_b1, g_W2, g_b2):
    n, vin = x.shape
    e = edge_index.shape[1]
    ein = edge_attr.shape[1]
    uin = u.shape[1]
    h = e_W2.shape[1]

    row = edge_index[0]
    col = edge_index[1]
    w1s = e_W1[:vin]
    w1d = e_W1[vin:2 * vin]
    w1e = e_W1[2 * vin:2 * vin + ein]
    w1u = e_W1[2 * vin + ein:]
    u2 = u.reshape(1, uin)
    eb1 = e_b1.reshape(1, h)
    eb2 = e_b2.reshape(1, h)

    xs, xd = _prep(x, u2, w1s, w1d, w1u, eb1, 2000)
    s_e = _sc_gather_add(xs, xd, row, col)
    ne, esum = _edge_mlp(s_e, edge_attr, w1e, e_W2, eb2, 2000)
    aggp = _sc_scatter_add(ne, col, n)

    mkx = Mk[:, :vin].T
    mka = Mk[:, vin:vin + h].T
    mku = Mk[:, vin + h:].T
    logits = _logits(x, aggp[0], aggp[1], u2, mkx, mka, mku, 2000)
    mcol, scol = _softmax_stats(logits, 2000)
    new_x, new_u = _node_global_call(
        logits, mcol, scol, Mv, n_W1, n_b1.reshape(1, h), n_W2,
        n_b2.reshape(1, h), u2, esum, g_W1, g_b1.reshape(1, h), g_W2,
        g_b2.reshape(1, h), 2000, e)
    return (new_x, ne, new_u)


# speedup vs baseline: 5.9735x; 1.3313x over previous
import jax
import jax.numpy as jnp
from jax import lax
from jax.experimental import pallas as pl
from jax.experimental.pallas import tpu as pltpu
from jax.experimental.pallas import tpu_sc as plsc

_NC = 2
_NS = 16
_NW = _NC * _NS
_LANES = 16
_CHUNK = 80
_SLOTS = 5
_RC = 80


def _prep_body(x_ref, u_ref, w1s_ref, w1d_ref, w1u_ref, eb1_ref, xs_ref, xd_ref):
    c0 = jnp.dot(u_ref[...], w1u_ref[...], preferred_element_type=jnp.float32) + eb1_ref[...]
    xs = jnp.dot(x_ref[...], w1s_ref[...], preferred_element_type=jnp.float32) + c0
    xd = jnp.dot(x_ref[...], w1d_ref[...], preferred_element_type=jnp.float32)
    hw = xs.shape[1] // 2
    xs_ref[...] = pltpu.pack_elementwise(
        [xs[:, :hw], xs[:, hw:]], packed_dtype=jnp.bfloat16)
    xd_ref[...] = pltpu.pack_elementwise(
        [xd[:, :hw], xd[:, hw:]], packed_dtype=jnp.bfloat16)


def _prep(x, u2, w1s, w1d, w1u, eb1, bn):
    n, vin = x.shape
    h = w1s.shape[1]
    return pl.pallas_call(
        _prep_body,
        grid=(n // bn,),
        in_specs=[
            pl.BlockSpec((bn, vin), lambda i: (i, 0)),
            pl.BlockSpec(u2.shape, lambda i: (0, 0)),
            pl.BlockSpec(w1s.shape, lambda i: (0, 0)),
            pl.BlockSpec(w1d.shape, lambda i: (0, 0)),
            pl.BlockSpec(w1u.shape, lambda i: (0, 0)),
            pl.BlockSpec(eb1.shape, lambda i: (0, 0)),
        ],
        out_specs=[pl.BlockSpec((bn, h // 2), lambda i: (i, 0)),
                   pl.BlockSpec((bn, h // 2), lambda i: (i, 0))],
        out_shape=[jax.ShapeDtypeStruct((n, h // 2), jnp.int32),
                   jax.ShapeDtypeStruct((n, h // 2), jnp.int32)],
    )(x, u2, w1s, w1d, w1u, eb1)


_GSLOTS = 2


def _sc_gather_add(xs_p, xd_p, ei_flat):
    e = ei_flat.shape[0] // 2
    n, hw = xs_p.shape
    ept = e // _NS
    nchunks = ept // _CHUNK
    ngroups = nchunks // _GSLOTS
    nrch = n // _RC
    per = (nrch + _NS - 1) // _NS
    mesh = plsc.VectorSubcoreMesh(core_axis_name="c", subcore_axis_name="s")

    def body(xs_hbm, xd_hbm, ei_hbm, out_hbm, *scr):
        iv = scr[0:_GSLOTS]
        av = scr[_GSLOTS:2 * _GSLOTS]
        tb_sh = scr[2 * _GSLOTS]
        isem = scr[2 * _GSLOTS + 1:2 * _GSLOTS + 1 + _GSLOTS]
        gsem = scr[2 * _GSLOTS + 1 + _GSLOTS:2 * _GSLOTS + 1 + 2 * _GSLOTS]
        wsem = scr[2 * _GSLOTS + 1 + 2 * _GSLOTS]

        cid = lax.axis_index("c")
        sid = lax.axis_index("s")

        def pipeline(tbl_hbm, op):
            for t in range(per):
                cix = sid + _NS * t

                @pl.when(cix < nrch)
                def _(cix=cix):
                    r0 = pl.multiple_of(cix * _RC, 8)
                    pltpu.sync_copy(tbl_hbm.at[pl.ds(r0, _RC)], av[0])
                    pltpu.sync_copy(av[0], tb_sh.at[pl.ds(r0, _RC)])

            plsc.subcore_barrier()

            def group(g, carry):
                iobjs = []
                for s in range(_GSLOTS):
                    base = pl.multiple_of(sid * ept + (g * _GSLOTS + s) * _CHUNK, 8)
                    ibase = pl.multiple_of(op * e + base, 8)
                    iobjs.append((base, pltpu.async_copy(
                        ei_hbm.at[pl.ds(ibase, _CHUNK)], iv[s], isem[s])))
                gobjs = []
                for s in range(_GSLOTS):
                    base, o1 = iobjs[s]
                    o1.wait()
                    gobjs.append((base, pltpu.async_copy(tb_sh.at[iv[s]], av[s], gsem[s])))
                wobjs = []
                for s in range(_GSLOTS):
                    base, ga = gobjs[s]
                    ga.wait()
                    wobjs.append(pltpu.async_copy(
                        av[s], out_hbm.at[op, pl.ds(base, _CHUNK)], wsem))
                for w in wobjs:
                    w.wait()
                return carry

            lax.fori_loop(0, ngroups, group, 0)

        @pl.when(cid == 0)
        def _():
            pipeline(xs_hbm, 0)

        @pl.when(cid == 1)
        def _():
            pipeline(xd_hbm, 1)

    f = pl.kernel(
        body,
        out_type=jax.ShapeDtypeStruct((_NC, e, hw), jnp.int32),
        mesh=mesh,
        scratch_types=(
            [pltpu.VMEM((_CHUNK,), jnp.int32) for _ in range(_GSLOTS)]
            + [pltpu.VMEM((_CHUNK, hw), jnp.int32) for _ in range(_GSLOTS)]
            + [pltpu.VMEM_SHARED((n, hw), jnp.int32)]
            + [pltpu.SemaphoreType.DMA for _ in range(2 * _GSLOTS)]
            + [pltpu.SemaphoreType.DMA]
        ),
    )
    return f(xs_p, xd_p, ei_flat)


def _edge_body(sa_ref, sb_ref, ea_ref, w1e_ref, w2_ref, eb2_ref, ne_ref, esum_ref, acc_ref):
    i = pl.program_id(0)
    sa = sa_ref[0]
    sb = sb_ref[0]

    def _unpack(p, idx):
        return pltpu.unpack_elementwise(
            p, index=idx, packed_dtype=jnp.bfloat16, unpacked_dtype=jnp.float32)

    lo = _unpack(sa, 0) + _unpack(sb, 0)
    hi = _unpack(sa, 1) + _unpack(sb, 1)
    s = jnp.concatenate([lo, hi], axis=1)
    pre = s + jnp.dot(ea_ref[...], w1e_ref[...], preferred_element_type=jnp.float32)
    hh = jnp.maximum(pre, 0.0)
    ne = jnp.dot(hh, w2_ref[...], preferred_element_type=jnp.float32) + eb2_ref[...]
    ne_ref[...] = ne

    @pl.when(i == 0)
    def _():
        acc_ref[...] = jnp.zeros_like(acc_ref)

    acc_ref[...] += jnp.sum(ne, axis=0, keepdims=True)

    @pl.when(i == pl.num_programs(0) - 1)
    def _():
        esum_ref[...] = acc_ref[...]


def _edge_mlp(s_ab, ea, w1e, w2, eb2, be):
    _, e, hw = s_ab.shape
    h = w2.shape[1]
    ein = ea.shape[1]
    return pl.pallas_call(
        _edge_body,
        grid=(e // be,),
        in_specs=[
            pl.BlockSpec((1, be, hw), lambda i: (0, i, 0)),
            pl.BlockSpec((1, be, hw), lambda i: (1, i, 0)),
            pl.BlockSpec((be, ein), lambda i: (i, 0)),
            pl.BlockSpec(w1e.shape, lambda i: (0, 0)),
            pl.BlockSpec(w2.shape, lambda i: (0, 0)),
            pl.BlockSpec(eb2.shape, lambda i: (0, 0)),
        ],
        out_specs=[pl.BlockSpec((be, h), lambda i: (i, 0)),
                   pl.BlockSpec((1, h), lambda i: (0, 0))],
        out_shape=[jax.ShapeDtypeStruct((e, h), jnp.float32),
                   jax.ShapeDtypeStruct((1, h), jnp.float32)],
        scratch_shapes=[pltpu.VMEM((1, h), jnp.float32)],
    )(s_ab, s_ab, ea, w1e, w2, eb2)


_SSLOTS = 1


def _sc_scatter_add(ne, col, n):
    e, h = ne.shape
    epw = e // _NW
    nchunks = epw // _CHUNK
    ngroups = nchunks // _SSLOTS
    nrch = n // _RC
    per = (nrch + _NS - 1) // _NS
    mesh = plsc.VectorSubcoreMesh(core_axis_name="c", subcore_axis_name="s")

    def body(ne_hbm, col_hbm, out_hbm, *scr):
        iv = scr[0:_SSLOTS]
        nv = scr[_SSLOTS:2 * _SSLOTS]
        z_v = scr[2 * _SSLOTS]
        agg_sh = scr[2 * _SSLOTS + 1]
        isem = scr[2 * _SSLOTS + 2:2 * _SSLOTS + 2 + _SSLOTS]
        nsem = scr[2 * _SSLOTS + 2 + _SSLOTS:2 * _SSLOTS + 2 + 2 * _SSLOTS]
        ssem = scr[2 * _SSLOTS + 2 + 2 * _SSLOTS]

        cid = lax.axis_index("c")
        sid = lax.axis_index("s")
        wid = sid * _NC + cid

        def zrow(j, carry):
            for q in range(h // _LANES):
                z_v[j, pl.ds(q * _LANES, _LANES)] = jnp.zeros((_LANES,), jnp.float32)
            return carry

        lax.fori_loop(0, _RC, zrow, 0)
        for t in range(per):
            cix = sid + _NS * t

            @pl.when(cix < nrch)
            def _(cix=cix):
                r0 = pl.multiple_of(cix * _RC, 8)
                pltpu.sync_copy(z_v, agg_sh.at[pl.ds(r0, _RC)])

        plsc.subcore_barrier()

        def group(g, carry):
            lobjs = []
            for s in range(_SSLOTS):
                base = pl.multiple_of(wid * epw + (g * _SSLOTS + s) * _CHUNK, 8)
                o1 = pltpu.async_copy(col_hbm.at[pl.ds(base, _CHUNK)], iv[s], isem[s])
                o2 = pltpu.async_copy(ne_hbm.at[pl.ds(base, _CHUNK)], nv[s], nsem[s])
                lobjs.append((o1, o2))
            for s in range(_SSLOTS):
                o1, o2 = lobjs[s]
                o1.wait()
                o2.wait()
                pltpu.sync_copy(nv[s], agg_sh.at[iv[s]], add=True)
            return carry

        lax.fori_loop(0, ngroups, group, 0)
        plsc.subcore_barrier()
        for t in range(per):
            cix = sid + _NS * t

            @pl.when(cix < nrch)
            def _(cix=cix):
                r0 = pl.multiple_of(cix * _RC, 8)
                pltpu.sync_copy(agg_sh.at[pl.ds(r0, _RC)], z_v)
                pltpu.sync_copy(z_v, out_hbm.at[cid, pl.ds(r0, _RC)])

    f = pl.kernel(
        body,
        out_type=jax.ShapeDtypeStruct((_NC, n, h), jnp.float32),
        mesh=mesh,
        scratch_types=(
            [pltpu.VMEM((_CHUNK,), jnp.int32) for _ in range(_SSLOTS)]
            + [pltpu.VMEM((_CHUNK, h), jnp.float32) for _ in range(_SSLOTS)]
            + [pltpu.VMEM((_RC, h), jnp.float32)]
            + [pltpu.VMEM_SHARED((n, h), jnp.float32)]
            + [pltpu.SemaphoreType.DMA for _ in range(2 * _SSLOTS)]
            + [pltpu.SemaphoreType.DMA]
        ),
    )
    return f(ne, col)


def _logits_body(x_ref, a0_ref, a1_ref, u_ref, mkx_ref, mka_ref, mku_ref, l_ref):
    agg = a0_ref[...] + a1_ref[...]
    l_ref[...] = (
        jnp.dot(x_ref[...], mkx_ref[...], preferred_element_type=jnp.float32)
        + jnp.dot(agg, mka_ref[...], preferred_element_type=jnp.float32)
        + jnp.dot(u_ref[...], mku_ref[...], preferred_element_type=jnp.float32)
    )


def _logits(x, a0, a1, u2, mkx, mka, mku, bn):
    n, vin = x.shape
    h = a0.shape[1]
    m = mkx.shape[1]
    return pl.pallas_call(
        _logits_body,
        grid=(n // bn,),
        in_specs=[
            pl.BlockSpec((bn, vin), lambda i: (i, 0)),
            pl.BlockSpec((bn, h), lambda i: (i, 0)),
            pl.BlockSpec((bn, h), lambda i: (i, 0)),
            pl.BlockSpec(u2.shape, lambda i: (0, 0)),
            pl.BlockSpec(mkx.shape, lambda i: (0, 0)),
            pl.BlockSpec(mka.shape, lambda i: (0, 0)),
            pl.BlockSpec(mku.shape, lambda i: (0, 0)),
        ],
        out_specs=pl.BlockSpec((bn, m), lambda i: (i, 0)),
        out_shape=jax.ShapeDtypeStruct((n, m), jnp.float32),
    )(x, a0, a1, u2, mkx, mka, mku)


def _stats_body(l_ref, m_ref, s_ref, macc_ref, sacc_ref):
    i = pl.program_id(0)

    @pl.when(i == 0)
    def _():
        macc_ref[...] = jnp.full_like(macc_ref, -1e30)
        sacc_ref[...] = jnp.zeros_like(sacc_ref)

    l = l_ref[...]
    bm = jnp.max(l, axis=0, keepdims=True)
    mold = macc_ref[...]
    mnew = jnp.maximum(mold, bm)
    sacc_ref[...] = sacc_ref[...] * jnp.exp(mold - mnew) + jnp.sum(
        jnp.exp(l - mnew), axis=0, keepdims=True)
    macc_ref[...] = mnew

    @pl.when(i == pl.num_programs(0) - 1)
    def _():
        m_ref[...] = macc_ref[...]
        s_ref[...] = sacc_ref[...]


def _softmax_stats(logits, bn):
    n, m = logits.shape
    return pl.pallas_call(
        _stats_body,
        grid=(n // bn,),
        in_specs=[pl.BlockSpec((bn, m), lambda i: (i, 0))],
        out_specs=[pl.BlockSpec((1, m), lambda i: (0, 0)),
                   pl.BlockSpec((1, m), lambda i: (0, 0))],
        out_shape=[jax.ShapeDtypeStruct((1, m), jnp.float32),
                   jax.ShapeDtypeStruct((1, m), jnp.float32)],
        scratch_shapes=[pltpu.VMEM((1, m), jnp.float32),
                        pltpu.VMEM((1, m), jnp.float32)],
    )(logits)


def _node_body(n_nodes, n_edges,
               l_ref, m_ref, s_ref, mv_ref, nw1_ref, nb1_ref, nw2_ref, nb2_ref,
               u_ref, esum_ref, gw1_ref, gb1_ref, gw2_ref, gb2_ref,
               nx_ref, nu_ref, acc_ref):
    i = pl.program_id(0)
    attn = jnp.exp(l_ref[...] - m_ref[...]) / s_ref[...]
    attn = attn / jnp.sum(attn, axis=1, keepdims=True)
    ao = jnp.dot(attn, mv_ref[...], preferred_element_type=jnp.float32)
    hn = jnp.maximum(
        jnp.dot(ao, nw1_ref[...], preferred_element_type=jnp.float32) + nb1_ref[...], 0.0)
    nx = jnp.dot(hn, nw2_ref[...], preferred_element_type=jnp.float32) + nb2_ref[...]
    nx_ref[...] = nx

    @pl.when(i == 0)
    def _():
        acc_ref[...] = jnp.zeros_like(acc_ref)

    acc_ref[...] += jnp.sum(nx, axis=0, keepdims=True)

    @pl.when(i == pl.num_programs(0) - 1)
    def _():
        node_agg = acc_ref[...] * (1.0 / n_nodes)
        edge_agg = esum_ref[...] * (1.0 / n_edges)
        gcat = jnp.concatenate([u_ref[...], node_agg, edge_agg], axis=1)
        gh = jnp.maximum(
            jnp.dot(gcat, gw1_ref[...], preferred_element_type=jnp.float32) + gb1_ref[...], 0.0)
        nu_ref[...] = jnp.dot(gh, gw2_ref[...], preferred_element_type=jnp.float32) + gb2_ref[...]


def _node_global_call(logits, mcol, scol, mv, nw1, nb1, nw2, nb2, u2, esum,
                      gw1, gb1, gw2, gb2, bn, n_edges):
    import functools
    n, m = logits.shape
    h = nw2.shape[1]
    body = functools.partial(_node_body, float(n), float(n_edges))
    full = lambda a: pl.BlockSpec(a.shape, lambda i: tuple(0 for _ in a.shape))
    return pl.pallas_call(
        body,
        grid=(n // bn,),
        in_specs=[
            pl.BlockSpec((bn, m), lambda i: (i, 0)),
            full(mcol), full(scol), full(mv), full(nw1), full(nb1),
            full(nw2), full(nb2), full(u2), full(esum), full(gw1),
            full(gb1), full(gw2), full(gb2),
        ],
        out_specs=[pl.BlockSpec((bn, h), lambda i: (i, 0)),
                   pl.BlockSpec((1, h), lambda i: (0, 0))],
        out_shape=[jax.ShapeDtypeStruct((n, h), jnp.float32),
                   jax.ShapeDtypeStruct((1, h), jnp.float32)],
        scratch_shapes=[pltpu.VMEM((1, h), jnp.float32)],
    )(logits, mcol, scol, mv, nw1, nb1, nw2, nb2, u2, esum, gw1, gb1, gw2, gb2)


def kernel(x, edge_index, edge_attr, u, batch,
           e_W1, e_b1, e_W2, e_b2, Mk, Mv,
           n_W1, n_b1, n_W2, n_b2, g_W1, g_b1, g_W2, g_b2):
    n, vin = x.shape
    e = edge_index.shape[1]
    ein = edge_attr.shape[1]
    uin = u.shape[1]
    h = e_W2.shape[1]

    row = edge_index[0]
    col = edge_index[1]
    w1s = e_W1[:vin]
    w1d = e_W1[vin:2 * vin]
    w1e = e_W1[2 * vin:2 * vin + ein]
    w1u = e_W1[2 * vin + ein:]
    u2 = u.reshape(1, uin)
    eb1 = e_b1.reshape(1, h)
    eb2 = e_b2.reshape(1, h)

    xs_p, xd_p = _prep(x, u2, w1s, w1d, w1u, eb1, 2000)
    s_ab = _sc_gather_add(xs_p, xd_p, edge_index.reshape(2 * e))
    ne, esum = _edge_mlp(s_ab, edge_attr, w1e, e_W2, eb2, 2000)
    aggp = _sc_scatter_add(ne, col, n)

    mkx = Mk[:, :vin].T
    mka = Mk[:, vin:vin + h].T
    mku = Mk[:, vin + h:].T
    logits = _logits(x, aggp[0], aggp[1], u2, mkx, mka, mku, 2000)
    mcol, scol = _softmax_stats(logits, 2000)
    new_x, new_u = _node_global_call(
        logits, mcol, scol, Mv, n_W1, n_b1.reshape(1, h), n_W2,
        n_b2.reshape(1, h), u2, esum, g_W1, g_b1.reshape(1, h), g_W2,
        g_b2.reshape(1, h), 2000, e)
    return (new_x, ne, new_u)


# speedup vs baseline: 6.2221x; 1.0416x over previous
import jax
import jax.numpy as jnp
from jax import lax
from jax.experimental import pallas as pl
from jax.experimental.pallas import tpu as pltpu
from jax.experimental.pallas import tpu_sc as plsc

_NC = 2
_NS = 16
_NW = _NC * _NS
_LANES = 16
_CHUNK = 80


def _prep_body(x_ref, u_ref, w1s_ref, w1d_ref, w1u_ref, eb1_ref, xs_ref, xd_ref):
    c0 = jnp.dot(u_ref[...], w1u_ref[...], preferred_element_type=jnp.float32) + eb1_ref[...]
    xs_ref[...] = jnp.dot(x_ref[...], w1s_ref[...], preferred_element_type=jnp.float32) + c0
    xd_ref[...] = jnp.dot(x_ref[...], w1d_ref[...], preferred_element_type=jnp.float32)


def _prep(x, u2, w1s, w1d, w1u, eb1, bn):
    n, vin = x.shape
    h = w1s.shape[1]
    return pl.pallas_call(
        _prep_body,
        grid=(n // bn,),
        in_specs=[
            pl.BlockSpec((bn, vin), lambda i: (i, 0)),
            pl.BlockSpec(u2.shape, lambda i: (0, 0)),
            pl.BlockSpec(w1s.shape, lambda i: (0, 0)),
            pl.BlockSpec(w1d.shape, lambda i: (0, 0)),
            pl.BlockSpec(w1u.shape, lambda i: (0, 0)),
            pl.BlockSpec(eb1.shape, lambda i: (0, 0)),
        ],
        out_specs=[pl.BlockSpec((bn, h), lambda i: (i, 0)),
                   pl.BlockSpec((bn, h), lambda i: (i, 0))],
        out_shape=[jax.ShapeDtypeStruct((n, h), jnp.float32),
                   jax.ShapeDtypeStruct((n, h), jnp.float32)],
    )(x, u2, w1s, w1d, w1u, eb1)


_GSLOTS = 3


def _sc_gather_add(xs, xd, row, col):
    e = row.shape[0]
    h = xs.shape[1]
    epw = e // _NW
    nchunks = epw // _CHUNK
    ngroups = nchunks // _GSLOTS
    ntail = nchunks - ngroups * _GSLOTS
    mesh = plsc.VectorSubcoreMesh(core_axis_name="c", subcore_axis_name="s")

    def body(xs_hbm, xd_hbm, row_hbm, col_hbm, s_hbm, *scr):
        ir = scr[0:_GSLOTS]
        ic = scr[_GSLOTS:2 * _GSLOTS]
        av = scr[2 * _GSLOTS:3 * _GSLOTS]
        bv = scr[3 * _GSLOTS:4 * _GSLOTS]
        isem = scr[4 * _GSLOTS:5 * _GSLOTS]
        gsem = scr[5 * _GSLOTS:6 * _GSLOTS]
        wsem = scr[6 * _GSLOTS]
        wid = lax.axis_index("s") * _NC + lax.axis_index("c")

        def run_chunks(k0, nslots):
            iobjs = []
            for s in range(nslots):
                base = pl.multiple_of(wid * epw + (k0 + s) * _CHUNK, 8)
                o1 = pltpu.async_copy(row_hbm.at[pl.ds(base, _CHUNK)], ir[s], isem[s])
                o2 = pltpu.async_copy(col_hbm.at[pl.ds(base, _CHUNK)], ic[s], isem[s])
                iobjs.append((base, o1, o2))
            gobjs = []
            for s in range(nslots):
                base, o1, o2 = iobjs[s]
                o1.wait()
                o2.wait()
                ga = pltpu.async_copy(xs_hbm.at[ir[s]], av[s], gsem[s])
                gb = pltpu.async_copy(xd_hbm.at[ic[s]], bv[s], gsem[s])
                gobjs.append((base, ga, gb))
            wobjs = []
            for s in range(nslots):
                base, ga, gb = gobjs[s]
                ga.wait()
                gb.wait()

                def addrow(j, c2, s=s):
                    for q in range(h // _LANES):
                        sl = pl.ds(q * _LANES, _LANES)
                        av[s][j, sl] = av[s][j, sl] + bv[s][j, sl]
                    return c2

                lax.fori_loop(0, _CHUNK, addrow, 0)
                wobjs.append(pltpu.async_copy(av[s], s_hbm.at[pl.ds(base, _CHUNK)], wsem))
            for w in wobjs:
                w.wait()

        def group(g, carry):
            run_chunks(g * _GSLOTS, _GSLOTS)
            return carry

        lax.fori_loop(0, ngroups, group, 0)
        if ntail:
            run_chunks(ngroups * _GSLOTS, ntail)

    f = pl.kernel(
        body,
        out_type=jax.ShapeDtypeStruct((e, h), jnp.float32),
        mesh=mesh,
        scratch_types=(
            [pltpu.VMEM((_CHUNK,), jnp.int32) for _ in range(2 * _GSLOTS)]
            + [pltpu.VMEM((_CHUNK, h), jnp.float32) for _ in range(2 * _GSLOTS)]
            + [pltpu.SemaphoreType.DMA for _ in range(2 * _GSLOTS)]
            + [pltpu.SemaphoreType.DMA]
        ),
    )
    return f(xs, xd, row, col)


def _edge_body(s_ref, ea_ref, w1e_ref, w2_ref, eb2_ref, ne_ref, esum_ref, acc_ref):
    i = pl.program_id(0)
    pre = s_ref[...] + jnp.dot(ea_ref[...], w1e_ref[...], preferred_element_type=jnp.float32)
    hh = jnp.maximum(pre, 0.0)
    ne = jnp.dot(hh, w2_ref[...], preferred_element_type=jnp.float32) + eb2_ref[...]
    ne_ref[...] = ne

    @pl.when(i == 0)
    def _():
        acc_ref[...] = jnp.zeros_like(acc_ref)

    acc_ref[...] += jnp.sum(ne, axis=0, keepdims=True)

    @pl.when(i == pl.num_programs(0) - 1)
    def _():
        esum_ref[...] = acc_ref[...]


def _edge_mlp(s_e, ea, w1e, w2, eb2, be):
    e, h = s_e.shape
    ein = ea.shape[1]
    return pl.pallas_call(
        _edge_body,
        grid=(e // be,),
        in_specs=[
            pl.BlockSpec((be, h), lambda i: (i, 0)),
            pl.BlockSpec((be, ein), lambda i: (i, 0)),
            pl.BlockSpec(w1e.shape, lambda i: (0, 0)),
            pl.BlockSpec(w2.shape, lambda i: (0, 0)),
            pl.BlockSpec(eb2.shape, lambda i: (0, 0)),
        ],
        out_specs=[pl.BlockSpec((be, h), lambda i: (i, 0)),
                   pl.BlockSpec((1, h), lambda i: (0, 0))],
        out_shape=[jax.ShapeDtypeStruct((e, h), jnp.float32),
                   jax.ShapeDtypeStruct((1, h), jnp.float32)],
        scratch_shapes=[pltpu.VMEM((1, h), jnp.float32)],
    )(s_e, ea, w1e, w2, eb2)


_SSLOTS = 2


def _sc_scatter_add(ne, col, n):
    e, h = ne.shape
    epw = e // _NW
    nchunks = epw // _CHUNK
    rc = _CHUNK
    nrch = n // rc
    per = (nrch + _NS - 1) // _NS
    mesh = plsc.VectorSubcoreMesh(core_axis_name="c", subcore_axis_name="s")

    ngroups = nchunks // _SSLOTS
    ntail = nchunks - ngroups * _SSLOTS

    def body(ne_hbm, col_hbm, out_hbm, *scr):
        iv = scr[0:_SSLOTS]
        nv = scr[_SSLOTS:2 * _SSLOTS]
        agg_sh = scr[2 * _SSLOTS]
        isem = scr[2 * _SSLOTS + 1:2 * _SSLOTS + 1 + _SSLOTS]
        nsem = scr[2 * _SSLOTS + 1 + _SSLOTS:2 * _SSLOTS + 1 + 2 * _SSLOTS]

        cid = lax.axis_index("c")
        sid = lax.axis_index("s")
        wid = sid * _NC + cid

        def zrow(j, carry):
            for q in range(h // _LANES):
                nv[0][j, pl.ds(q * _LANES, _LANES)] = jnp.zeros((_LANES,), jnp.float32)
            return carry

        lax.fori_loop(0, rc, zrow, 0)
        for t in range(per):
            cix = sid + _NS * t

            @pl.when(cix < nrch)
            def _(cix=cix):
                r0 = pl.multiple_of(cix * rc, 8)
                pltpu.sync_copy(nv[0], agg_sh.at[pl.ds(r0, rc)])

        plsc.subcore_barrier()

        def run_chunks(k0, nslots):
            lobjs = []
            for s in range(nslots):
                base = pl.multiple_of(wid * epw + (k0 + s) * _CHUNK, 8)
                o1 = pltpu.async_copy(col_hbm.at[pl.ds(base, _CHUNK)], iv[s], isem[s])
                o2 = pltpu.async_copy(ne_hbm.at[pl.ds(base, _CHUNK)], nv[s], nsem[s])
                lobjs.append((o1, o2))
            for s in range(nslots):
                o1, o2 = lobjs[s]
                o1.wait()
                o2.wait()
                pltpu.sync_copy(nv[s], agg_sh.at[iv[s]], add=True)

        def group(g, carry):
            run_chunks(g * _SSLOTS, _SSLOTS)
            return carry

        lax.fori_loop(0, ngroups, group, 0)
        if ntail:
            run_chunks(ngroups * _SSLOTS, ntail)
        plsc.subcore_barrier()
        for t in range(per):
            cix = sid + _NS * t

            @pl.when(cix < nrch)
            def _(cix=cix):
                r0 = pl.multiple_of(cix * rc, 8)
                pltpu.sync_copy(agg_sh.at[pl.ds(r0, rc)], nv[0])
                pltpu.sync_copy(nv[0], out_hbm.at[cid, pl.ds(r0, rc)])

    f = pl.kernel(
        body,
        out_type=jax.ShapeDtypeStruct((_NC, n, h), jnp.float32),
        mesh=mesh,
        scratch_types=(
            [pltpu.VMEM((_CHUNK,), jnp.int32) for _ in range(_SSLOTS)]
            + [pltpu.VMEM((_CHUNK, h), jnp.float32) for _ in range(_SSLOTS)]
            + [pltpu.VMEM_SHARED((n, h), jnp.float32)]
            + [pltpu.SemaphoreType.DMA for _ in range(2 * _SSLOTS)]
        ),
    )
    return f(ne, col)


def _logits_body(x_ref, a0_ref, a1_ref, u_ref, mkx_ref, mka_ref, mku_ref, l_ref):
    agg = a0_ref[...] + a1_ref[...]
    l_ref[...] = (
        jnp.dot(x_ref[...], mkx_ref[...], preferred_element_type=jnp.float32)
        + jnp.dot(agg, mka_ref[...], preferred_element_type=jnp.float32)
        + jnp.dot(u_ref[...], mku_ref[...], preferred_element_type=jnp.float32)
    )


def _logits(x, a0, a1, u2, mkx, mka, mku, bn):
    n, vin = x.shape
    h = a0.shape[1]
    m = mkx.shape[1]
    return pl.pallas_call(
        _logits_body,
        grid=(n // bn,),
        in_specs=[
            pl.BlockSpec((bn, vin), lambda i: (i, 0)),
            pl.BlockSpec((bn, h), lambda i: (i, 0)),
            pl.BlockSpec((bn, h), lambda i: (i, 0)),
            pl.BlockSpec(u2.shape, lambda i: (0, 0)),
            pl.BlockSpec(mkx.shape, lambda i: (0, 0)),
            pl.BlockSpec(mka.shape, lambda i: (0, 0)),
            pl.BlockSpec(mku.shape, lambda i: (0, 0)),
        ],
        out_specs=pl.BlockSpec((bn, m), lambda i: (i, 0)),
        out_shape=jax.ShapeDtypeStruct((n, m), jnp.float32),
    )(x, a0, a1, u2, mkx, mka, mku)


def _stats_body(l_ref, m_ref, s_ref, macc_ref, sacc_ref):
    i = pl.program_id(0)

    @pl.when(i == 0)
    def _():
        macc_ref[...] = jnp.full_like(macc_ref, -1e30)
        sacc_ref[...] = jnp.zeros_like(sacc_ref)

    l = l_ref[...]
    bm = jnp.max(l, axis=0, keepdims=True)
    mold = macc_ref[...]
    mnew = jnp.maximum(mold, bm)
    sacc_ref[...] = sacc_ref[...] * jnp.exp(mold - mnew) + jnp.sum(
        jnp.exp(l - mnew), axis=0, keepdims=True)
    macc_ref[...] = mnew

    @pl.when(i == pl.num_programs(0) - 1)
    def _():
        m_ref[...] = macc_ref[...]
        s_ref[...] = sacc_ref[...]


def _softmax_stats(logits, bn):
    n, m = logits.shape
    return pl.pallas_call(
        _stats_body,
        grid=(n // bn,),
        in_specs=[pl.BlockSpec((bn, m), lambda i: (i, 0))],
        out_specs=[pl.BlockSpec((1, m), lambda i: (0, 0)),
                   pl.BlockSpec((1, m), lambda i: (0, 0))],
        out_shape=[jax.ShapeDtypeStruct((1, m), jnp.float32),
                   jax.ShapeDtypeStruct((1, m), jnp.float32)],
        scratch_shapes=[pltpu.VMEM((1, m), jnp.float32),
                        pltpu.VMEM((1, m), jnp.float32)],
    )(logits)


def _node_body(n_nodes, n_edges,
               l_ref, m_ref, s_ref, mv_ref, nw1_ref, nb1_ref, nw2_ref, nb2_ref,
               u_ref, esum_ref, gw1_ref, gb1_ref, gw2_ref, gb2_ref,
               nx_ref, nu_ref, acc_ref):
    i = pl.program_id(0)
    attn = jnp.exp(l_ref[...] - m_ref[...]) / s_ref[...]
    attn = attn / jnp.sum(attn, axis=1, keepdims=True)
    ao = jnp.dot(attn, mv_ref[...], preferred_element_type=jnp.float32)
    hn = jnp.maximum(
        jnp.dot(ao, nw1_ref[...], preferred_element_type=jnp.float32) + nb1_ref[...], 0.0)
    nx = jnp.dot(hn, nw2_ref[...], preferred_element_type=jnp.float32) + nb2_ref[...]
    nx_ref[...] = nx

    @pl.when(i == 0)
    def _():
        acc_ref[...] = jnp.zeros_like(acc_ref)

    acc_ref[...] += jnp.sum(nx, axis=0, keepdims=True)

    @pl.when(i == pl.num_programs(0) - 1)
    def _():
        node_agg = acc_ref[...] * (1.0 / n_nodes)
        edge_agg = esum_ref[...] * (1.0 / n_edges)
        gcat = jnp.concatenate([u_ref[...], node_agg, edge_agg], axis=1)
        gh = jnp.maximum(
            jnp.dot(gcat, gw1_ref[...], preferred_element_type=jnp.float32) + gb1_ref[...], 0.0)
        nu_ref[...] = jnp.dot(gh, gw2_ref[...], preferred_element_type=jnp.float32) + gb2_ref[...]


def _node_global_call(logits, mcol, scol, mv, nw1, nb1, nw2, nb2, u2, esum,
                      gw1, gb1, gw2, gb2, bn, n_edges):
    import functools
    n, m = logits.shape
    h = nw2.shape[1]
    body = functools.partial(_node_body, float(n), float(n_edges))
    full = lambda a: pl.BlockSpec(a.shape, lambda i: tuple(0 for _ in a.shape))
    return pl.pallas_call(
        body,
        grid=(n // bn,),
        in_specs=[
            pl.BlockSpec((bn, m), lambda i: (i, 0)),
            full(mcol), full(scol), full(mv), full(nw1), full(nb1),
            full(nw2), full(nb2), full(u2), full(esum), full(gw1),
            full(gb1), full(gw2), full(gb2),
        ],
        out_specs=[pl.BlockSpec((bn, h), lambda i: (i, 0)),
                   pl.BlockSpec((1, h), lambda i: (0, 0))],
        out_shape=[jax.ShapeDtypeStruct((n, h), jnp.float32),
                   jax.ShapeDtypeStruct((1, h), jnp.float32)],
        scratch_shapes=[pltpu.VMEM((1, h), jnp.float32)],
    )(logits, mcol, scol, mv, nw1, nb1, nw2, nb2, u2, esum, gw1, gb1, gw2, gb2)


def kernel(x, edge_index, edge_attr, u, batch,
           e_W1, e_b1, e_W2, e_b2, Mk, Mv,
           n_W1, n_b1, n_W2, n_b2, g_W1, g_b1, g_W2, g_b2):
    n, vin = x.shape
    e = edge_index.shape[1]
    ein = edge_attr.shape[1]
    uin = u.shape[1]
    h = e_W2.shape[1]

    row = edge_index[0]
    col = edge_index[1]
    w1s = e_W1[:vin]
    w1d = e_W1[vin:2 * vin]
    w1e = e_W1[2 * vin:2 * vin + ein]
    w1u = e_W1[2 * vin + ein:]
    u2 = u.reshape(1, uin)
    eb1 = e_b1.reshape(1, h)
    eb2 = e_b2.reshape(1, h)

    xs, xd = _prep(x, u2, w1s, w1d, w1u, eb1, 2000)
    s_e = _sc_gather_add(xs, xd, row, col)
    ne, esum = _edge_mlp(s_e, edge_attr, w1e, e_W2, eb2, 2000)
    aggp = _sc_scatter_add(ne, col, n)

    mkx = Mk[:, :vin].T
    mka = Mk[:, vin:vin + h].T
    mku = Mk[:, vin + h:].T
    logits = _logits(x, aggp[0], aggp[1], u2, mkx, mka, mku, 2000)
    mcol, scol = _softmax_stats(logits, 2000)
    new_x, new_u = _node_global_call(
        logits, mcol, scol, Mv, n_W1, n_b1.reshape(1, h), n_W2,
        n_b2.reshape(1, h), u2, esum, g_W1, g_b1.reshape(1, h), g_W2,
        g_b2.reshape(1, h), 2000, e)
    return (new_x, ne, new_u)


# speedup vs baseline: 6.7056x; 1.0777x over previous
import jax
import jax.numpy as jnp
from jax import lax
from jax.experimental import pallas as pl
from jax.experimental.pallas import tpu as pltpu
from jax.experimental.pallas import tpu_sc as plsc

_NC = 2
_NS = 16
_NW = _NC * _NS
_LANES = 16
_CHUNK = 80


def _prep_body(x_ref, u_ref, w1s_ref, w1d_ref, w1u_ref, eb1_ref, xs_ref, xd_ref):
    c0 = jnp.dot(u_ref[...], w1u_ref[...], preferred_element_type=jnp.float32) + eb1_ref[...]
    xs_ref[...] = jnp.dot(x_ref[...], w1s_ref[...], preferred_element_type=jnp.float32) + c0
    xd_ref[...] = jnp.dot(x_ref[...], w1d_ref[...], preferred_element_type=jnp.float32)


def _prep(x, u2, w1s, w1d, w1u, eb1, bn):
    n, vin = x.shape
    h = w1s.shape[1]
    return pl.pallas_call(
        _prep_body,
        grid=(n // bn,),
        in_specs=[
            pl.BlockSpec((bn, vin), lambda i: (i, 0)),
            pl.BlockSpec(u2.shape, lambda i: (0, 0)),
            pl.BlockSpec(w1s.shape, lambda i: (0, 0)),
            pl.BlockSpec(w1d.shape, lambda i: (0, 0)),
            pl.BlockSpec(w1u.shape, lambda i: (0, 0)),
            pl.BlockSpec(eb1.shape, lambda i: (0, 0)),
        ],
        out_specs=[pl.BlockSpec((bn, h), lambda i: (i, 0)),
                   pl.BlockSpec((bn, h), lambda i: (i, 0))],
        out_shape=[jax.ShapeDtypeStruct((n, h), jnp.float32),
                   jax.ShapeDtypeStruct((n, h), jnp.float32)],
    )(x, u2, w1s, w1d, w1u, eb1)


_GSLOTS = 3


def _sc_gather_add(xs, xd, row, col):
    e = row.shape[0]
    h = xs.shape[1]
    epw = e // _NW
    nchunks = epw // _CHUNK
    ngroups = nchunks // _GSLOTS
    ntail = nchunks - ngroups * _GSLOTS
    mesh = plsc.VectorSubcoreMesh(core_axis_name="c", subcore_axis_name="s")

    def body(xs_hbm, xd_hbm, row_hbm, col_hbm, s_hbm, *scr):
        ir = scr[0:_GSLOTS]
        ic = scr[_GSLOTS:2 * _GSLOTS]
        av = scr[2 * _GSLOTS:3 * _GSLOTS]
        bv = scr[3 * _GSLOTS:4 * _GSLOTS]
        isem = scr[4 * _GSLOTS:5 * _GSLOTS]
        gsem = scr[5 * _GSLOTS:6 * _GSLOTS]
        wsem = scr[6 * _GSLOTS]
        wid = lax.axis_index("s") * _NC + lax.axis_index("c")

        def run_chunks(k0, nslots):
            iobjs = []
            for s in range(nslots):
                base = pl.multiple_of(wid * epw + (k0 + s) * _CHUNK, 8)
                o1 = pltpu.async_copy(row_hbm.at[pl.ds(base, _CHUNK)], ir[s], isem[s])
                o2 = pltpu.async_copy(col_hbm.at[pl.ds(base, _CHUNK)], ic[s], isem[s])
                iobjs.append((base, o1, o2))
            gobjs = []
            for s in range(nslots):
                base, o1, o2 = iobjs[s]
                o1.wait()
                o2.wait()
                ga = pltpu.async_copy(xs_hbm.at[ir[s]], av[s], gsem[s])
                gb = pltpu.async_copy(xd_hbm.at[ic[s]], bv[s], gsem[s])
                gobjs.append((base, ga, gb))
            wobjs = []
            for s in range(nslots):
                base, ga, gb = gobjs[s]
                ga.wait()
                gb.wait()

                def addrow(j, c2, s=s):
                    for q in range(h // _LANES):
                        sl = pl.ds(q * _LANES, _LANES)
                        av[s][j, sl] = av[s][j, sl] + bv[s][j, sl]
                    return c2

                lax.fori_loop(0, _CHUNK, addrow, 0)
                wobjs.append(pltpu.async_copy(av[s], s_hbm.at[pl.ds(base, _CHUNK)], wsem))
            for w in wobjs:
                w.wait()

        def group(g, carry):
            run_chunks(g * _GSLOTS, _GSLOTS)
            return carry

        lax.fori_loop(0, ngroups, group, 0)
        if ntail:
            run_chunks(ngroups * _GSLOTS, ntail)

    f = pl.kernel(
        body,
        out_type=jax.ShapeDtypeStruct((e, h), jnp.float32),
        mesh=mesh,
        scratch_types=(
            [pltpu.VMEM((_CHUNK,), jnp.int32) for _ in range(2 * _GSLOTS)]
            + [pltpu.VMEM((_CHUNK, h), jnp.float32) for _ in range(2 * _GSLOTS)]
            + [pltpu.SemaphoreType.DMA for _ in range(2 * _GSLOTS)]
            + [pltpu.SemaphoreType.DMA]
        ),
    )
    return f(xs, xd, row, col)


def _edge_body(s_ref, ea_ref, w1e_ref, w2_ref, eb2_ref, ne_ref, esum_ref, acc_ref):
    i = pl.program_id(0)
    pre = s_ref[...] + jnp.dot(ea_ref[...], w1e_ref[...], preferred_element_type=jnp.float32)
    hh = jnp.maximum(pre, 0.0)
    ne = jnp.dot(hh, w2_ref[...], preferred_element_type=jnp.float32) + eb2_ref[...]
    ne_ref[...] = ne

    @pl.when(i == 0)
    def _():
        acc_ref[...] = jnp.zeros_like(acc_ref)

    acc_ref[...] += jnp.sum(ne, axis=0, keepdims=True)

    @pl.when(i == pl.num_programs(0) - 1)
    def _():
        esum_ref[...] = acc_ref[...]


def _edge_mlp_first(s_e, ea, w1e, w2, eb2, be, e_total):
    eh, h = s_e.shape
    ein = ea.shape[1]
    return pl.pallas_call(
        _edge_body,
        grid=(eh // be,),
        in_specs=[
            pl.BlockSpec((be, h), lambda i: (i, 0)),
            pl.BlockSpec((be, ein), lambda i: (i, 0)),
            pl.BlockSpec(w1e.shape, lambda i: (0, 0)),
            pl.BlockSpec(w2.shape, lambda i: (0, 0)),
            pl.BlockSpec(eb2.shape, lambda i: (0, 0)),
        ],
        out_specs=[pl.BlockSpec((be, h), lambda i: (i, 0)),
                   pl.BlockSpec((1, h), lambda i: (0, 0))],
        out_shape=[jax.ShapeDtypeStruct((e_total, h), jnp.float32),
                   jax.ShapeDtypeStruct((1, h), jnp.float32)],
        scratch_shapes=[pltpu.VMEM((1, h), jnp.float32)],
    )(s_e, ea, w1e, w2, eb2)


def _edge_body_b(ne_in_ref, s_ref, ea_ref, w1e_ref, w2_ref, eb2_ref,
                 ne_ref, esum_ref, acc_ref):
    del ne_in_ref
    _edge_body(s_ref, ea_ref, w1e_ref, w2_ref, eb2_ref, ne_ref, esum_ref, acc_ref)


def _edge_mlp_second(ne_a, s_e, ea, w1e, w2, eb2, be, blk_off):
    eh, h = s_e.shape
    ein = ea.shape[1]
    e_total = ne_a.shape[0]
    return pl.pallas_call(
        _edge_body_b,
        grid=(eh // be,),
        in_specs=[
            pl.BlockSpec(memory_space=pltpu.MemorySpace.HBM),
            pl.BlockSpec((be, h), lambda i: (i, 0)),
            pl.BlockSpec((be, ein), lambda i: (i, 0)),
            pl.BlockSpec(w1e.shape, lambda i: (0, 0)),
            pl.BlockSpec(w2.shape, lambda i: (0, 0)),
            pl.BlockSpec(eb2.shape, lambda i: (0, 0)),
        ],
        out_specs=[pl.BlockSpec((be, h), lambda i, blk_off=blk_off: (i + blk_off, 0)),
                   pl.BlockSpec((1, h), lambda i: (0, 0))],
        out_shape=[jax.ShapeDtypeStruct((e_total, h), jnp.float32),
                   jax.ShapeDtypeStruct((1, h), jnp.float32)],
        scratch_shapes=[pltpu.VMEM((1, h), jnp.float32)],
        input_output_aliases={0: 0},
    )(ne_a, s_e, ea, w1e, w2, eb2)


_SSLOTS = 2


def _sc_scatter_add(ne, col, n, off):
    e = col.shape[0]
    h = ne.shape[1]
    epw = e // _NW
    nchunks = epw // _CHUNK
    rc = _CHUNK
    nrch = n // rc
    per = (nrch + _NS - 1) // _NS
    mesh = plsc.VectorSubcoreMesh(core_axis_name="c", subcore_axis_name="s")

    ngroups = nchunks // _SSLOTS
    ntail = nchunks - ngroups * _SSLOTS

    def body(ne_hbm, col_hbm, out_hbm, *scr):
        iv = scr[0:_SSLOTS]
        nv = scr[_SSLOTS:2 * _SSLOTS]
        agg_sh = scr[2 * _SSLOTS]
        isem = scr[2 * _SSLOTS + 1:2 * _SSLOTS + 1 + _SSLOTS]
        nsem = scr[2 * _SSLOTS + 1 + _SSLOTS:2 * _SSLOTS + 1 + 2 * _SSLOTS]

        cid = lax.axis_index("c")
        sid = lax.axis_index("s")
        wid = sid * _NC + cid

        def zrow(j, carry):
            for q in range(h // _LANES):
                nv[0][j, pl.ds(q * _LANES, _LANES)] = jnp.zeros((_LANES,), jnp.float32)
            return carry

        lax.fori_loop(0, rc, zrow, 0)
        for t in range(per):
            cix = sid + _NS * t

            @pl.when(cix < nrch)
            def _(cix=cix):
                r0 = pl.multiple_of(cix * rc, 8)
                pltpu.sync_copy(nv[0], agg_sh.at[pl.ds(r0, rc)])

        plsc.subcore_barrier()

        def run_chunks(k0, nslots):
            lobjs = []
            for s in range(nslots):
                base = pl.multiple_of(wid * epw + (k0 + s) * _CHUNK, 8)
                nbase = pl.multiple_of(off + base, 8)
                o1 = pltpu.async_copy(col_hbm.at[pl.ds(base, _CHUNK)], iv[s], isem[s])
                o2 = pltpu.async_copy(ne_hbm.at[pl.ds(nbase, _CHUNK)], nv[s], nsem[s])
                lobjs.append((o1, o2))
            for s in range(nslots):
                o1, o2 = lobjs[s]
                o1.wait()
                o2.wait()
                pltpu.sync_copy(nv[s], agg_sh.at[iv[s]], add=True)

        def group(g, carry):
            run_chunks(g * _SSLOTS, _SSLOTS)
            return carry

        lax.fori_loop(0, ngroups, group, 0)
        if ntail:
            run_chunks(ngroups * _SSLOTS, ntail)
        plsc.subcore_barrier()
        for t in range(per):
            cix = sid + _NS * t

            @pl.when(cix < nrch)
            def _(cix=cix):
                r0 = pl.multiple_of(cix * rc, 8)
                pltpu.sync_copy(agg_sh.at[pl.ds(r0, rc)], nv[0])
                pltpu.sync_copy(nv[0], out_hbm.at[cid, pl.ds(r0, rc)])

    f = pl.kernel(
        body,
        out_type=jax.ShapeDtypeStruct((_NC, n, h), jnp.float32),
        mesh=mesh,
        scratch_types=(
            [pltpu.VMEM((_CHUNK,), jnp.int32) for _ in range(_SSLOTS)]
            + [pltpu.VMEM((_CHUNK, h), jnp.float32) for _ in range(_SSLOTS)]
            + [pltpu.VMEM_SHARED((n, h), jnp.float32)]
            + [pltpu.SemaphoreType.DMA for _ in range(2 * _SSLOTS)]
        ),
    )
    return f(ne, col)


def _logits_body(x_ref, a0_ref, a1_ref, a2_ref, a3_ref, u_ref,
                 mkx_ref, mka_ref, mku_ref, l_ref):
    agg = (a0_ref[0] + a1_ref[0]) + (a2_ref[0] + a3_ref[0])
    l_ref[...] = (
        jnp.dot(x_ref[...], mkx_ref[...], preferred_element_type=jnp.float32)
        + jnp.dot(agg, mka_ref[...], preferred_element_type=jnp.float32)
        + jnp.dot(u_ref[...], mku_ref[...], preferred_element_type=jnp.float32)
    )


def _logits(x, aggp1, aggp2, u2, mkx, mka, mku, bn):
    n, vin = x.shape
    h = aggp1.shape[2]
    m = mkx.shape[1]
    pspec = lambda c: pl.BlockSpec((1, bn, h), lambda i, c=c: (c, i, 0))
    return pl.pallas_call(
        _logits_body,
        grid=(n // bn,),
        in_specs=[
            pl.BlockSpec((bn, vin), lambda i: (i, 0)),
            pspec(0), pspec(1), pspec(0), pspec(1),
            pl.BlockSpec(u2.shape, lambda i: (0, 0)),
            pl.BlockSpec(mkx.shape, lambda i: (0, 0)),
            pl.BlockSpec(mka.shape, lambda i: (0, 0)),
            pl.BlockSpec(mku.shape, lambda i: (0, 0)),
        ],
        out_specs=pl.BlockSpec((bn, m), lambda i: (i, 0)),
        out_shape=jax.ShapeDtypeStruct((n, m), jnp.float32),
    )(x, aggp1, aggp1, aggp2, aggp2, u2, mkx, mka, mku)


def _stats_body(l_ref, m_ref, s_ref, macc_ref, sacc_ref):
    i = pl.program_id(0)

    @pl.when(i == 0)
    def _():
        macc_ref[...] = jnp.full_like(macc_ref, -1e30)
        sacc_ref[...] = jnp.zeros_like(sacc_ref)

    l = l_ref[...]
    bm = jnp.max(l, axis=0, keepdims=True)
    mold = macc_ref[...]
    mnew = jnp.maximum(mold, bm)
    sacc_ref[...] = sacc_ref[...] * jnp.exp(mold - mnew) + jnp.sum(
        jnp.exp(l - mnew), axis=0, keepdims=True)
    macc_ref[...] = mnew

    @pl.when(i == pl.num_programs(0) - 1)
    def _():
        m_ref[...] = macc_ref[...]
        s_ref[...] = sacc_ref[...]


def _softmax_stats(logits, bn):
    n, m = logits.shape
    return pl.pallas_call(
        _stats_body,
        grid=(n // bn,),
        in_specs=[pl.BlockSpec((bn, m), lambda i: (i, 0))],
        out_specs=[pl.BlockSpec((1, m), lambda i: (0, 0)),
                   pl.BlockSpec((1, m), lambda i: (0, 0))],
        out_shape=[jax.ShapeDtypeStruct((1, m), jnp.float32),
                   jax.ShapeDtypeStruct((1, m), jnp.float32)],
        scratch_shapes=[pltpu.VMEM((1, m), jnp.float32),
                        pltpu.VMEM((1, m), jnp.float32)],
    )(logits)


def _node_body(n_nodes, n_edges,
               l_ref, m_ref, s_ref, mv_ref, nw1_ref, nb1_ref, nw2_ref, nb2_ref,
               u_ref, esum_ref, esum2_ref, gw1_ref, gb1_ref, gw2_ref, gb2_ref,
               nx_ref, nu_ref, acc_ref):
    i = pl.program_id(0)
    attn = jnp.exp(l_ref[...] - m_ref[...]) / s_ref[...]
    attn = attn / jnp.sum(attn, axis=1, keepdims=True)
    ao = jnp.dot(attn, mv_ref[...], preferred_element_type=jnp.float32)
    hn = jnp.maximum(
        jnp.dot(ao, nw1_ref[...], preferred_element_type=jnp.float32) + nb1_ref[...], 0.0)
    nx = jnp.dot(hn, nw2_ref[...], preferred_element_type=jnp.float32) + nb2_ref[...]
    nx_ref[...] = nx

    @pl.when(i == 0)
    def _():
        acc_ref[...] = jnp.zeros_like(acc_ref)

    acc_ref[...] += jnp.sum(nx, axis=0, keepdims=True)

    @pl.when(i == pl.num_programs(0) - 1)
    def _():
        node_agg = acc_ref[...] * (1.0 / n_nodes)
        edge_agg = (esum_ref[...] + esum2_ref[...]) * (1.0 / n_edges)
        gcat = jnp.concatenate([u_ref[...], node_agg, edge_agg], axis=1)
        gh = jnp.maximum(
            jnp.dot(gcat, gw1_ref[...], preferred_element_type=jnp.float32) + gb1_ref[...], 0.0)
        nu_ref[...] = jnp.dot(gh, gw2_ref[...], preferred_element_type=jnp.float32) + gb2_ref[...]


def _node_global_call(logits, mcol, scol, mv, nw1, nb1, nw2, nb2, u2, esum,
                      esum2, gw1, gb1, gw2, gb2, bn, n_edges):
    import functools
    n, m = logits.shape
    h = nw2.shape[1]
    body = functools.partial(_node_body, float(n), float(n_edges))
    full = lambda a: pl.BlockSpec(a.shape, lambda i: tuple(0 for _ in a.shape))
    return pl.pallas_call(
        body,
        grid=(n // bn,),
        in_specs=[
            pl.BlockSpec((bn, m), lambda i: (i, 0)),
            full(mcol), full(scol), full(mv), full(nw1), full(nb1),
            full(nw2), full(nb2), full(u2), full(esum), full(esum2),
            full(gw1), full(gb1), full(gw2), full(gb2),
        ],
        out_specs=[pl.BlockSpec((bn, h), lambda i: (i, 0)),
                   pl.BlockSpec((1, h), lambda i: (0, 0))],
        out_shape=[jax.ShapeDtypeStruct((n, h), jnp.float32),
                   jax.ShapeDtypeStruct((1, h), jnp.float32)],
        scratch_shapes=[pltpu.VMEM((1, h), jnp.float32)],
    )(logits, mcol, scol, mv, nw1, nb1, nw2, nb2, u2, esum, esum2,
      gw1, gb1, gw2, gb2)


def kernel(x, edge_index, edge_attr, u, batch,
           e_W1, e_b1, e_W2, e_b2, Mk, Mv,
           n_W1, n_b1, n_W2, n_b2, g_W1, g_b1, g_W2, g_b2):
    n, vin = x.shape
    e = edge_index.shape[1]
    ein = edge_attr.shape[1]
    uin = u.shape[1]
    h = e_W2.shape[1]

    row = edge_index[0]
    col = edge_index[1]
    w1s = e_W1[:vin]
    w1d = e_W1[vin:2 * vin]
    w1e = e_W1[2 * vin:2 * vin + ein]
    w1u = e_W1[2 * vin + ein:]
    u2 = u.reshape(1, uin)
    eb1 = e_b1.reshape(1, h)
    eb2 = e_b2.reshape(1, h)

    xs, xd = _prep(x, u2, w1s, w1d, w1u, eb1, 2000)

    e1 = (e * 3 // 5) // (_NW * _CHUNK) * (_NW * _CHUNK)
    be = 2000
    s1 = _sc_gather_add(xs, xd, row[:e1], col[:e1])
    s2 = _sc_gather_add(xs, xd, row[e1:], col[e1:])
    ne_a, esum1 = _edge_mlp_first(s1, edge_attr[:e1], w1e, e_W2, eb2, be, e)
    aggp1 = _sc_scatter_add(ne_a, col[:e1], n, 0)
    ne, esum2 = _edge_mlp_second(ne_a, s2, edge_attr[e1:], w1e, e_W2, eb2,
                                 be, e1 // be)
    aggp2 = _sc_scatter_add(ne, col[e1:], n, e1)

    mkx = Mk[:, :vin].T
    mka = Mk[:, vin:vin + h].T
    mku = Mk[:, vin + h:].T
    logits = _logits(x, aggp1, aggp2, u2, mkx, mka, mku, 2000)
    mcol, scol = _softmax_stats(logits, 2000)
    new_x, new_u = _node_global_call(
        logits, mcol, scol, Mv, n_W1, n_b1.reshape(1, h), n_W2,
        n_b2.reshape(1, h), u2, esum1, esum2, g_W1, g_b1.reshape(1, h), g_W2,
        g_b2.reshape(1, h), 2000, e)
    return (new_x, ne, new_u)
